# baseline (speedup 1.0000x reference)
import functools

import jax
import jax.numpy as jnp
from jax.experimental import pallas as pl
from jax.experimental.pallas import tpu as pltpu

SZ = 512
NT = 256
CHR = 16
NCH = SZ // CHR

_NEG_INF = float("-inf")


def _raster_kernel(td_ref, linr_ref, linc_ref,
                   ox_ref, oy_ref, oz_ref, oa_ref, zb_ref):
    py = linc_ref[...]
    iyf = jax.lax.broadcasted_iota(jnp.int32, (1, SZ), 1).astype(jnp.float32)

    zmin = td_ref[13, 0]
    zb_ref[...] = jnp.full((SZ, SZ), zmin, dtype=jnp.float32)

    def body(t, carry):
        v1x = td_ref[0, t]
        v1y = td_ref[1, t]
        v2x = td_ref[2, t]
        v2y = td_ref[3, t]
        v3x = td_ref[4, t]
        v3y = td_ref[5, t]
        az = td_ref[6, t]
        bz = td_ref[7, t]
        cz = td_ref[8, t]
        xminf = td_ref[9, t]
        xmaxf = td_ref[10, t]
        yminf = td_ref[11, t]
        ymaxf = td_ref[12, t]

        tyA = (py - v2y) * (v1x - v2x)
        tyB = (py - v3y) * (v2x - v3x)
        tyC = (py - v1y) * (v3x - v1x)
        colm = (iyf >= yminf) & (iyf < ymaxf)
        zc = jnp.where(colm, cz * py, _NEG_INF)

        rxA = (py - v2x) * (v1y - v2y)
        rxB = (py - v3x) * (v2y - v3y)
        rxC = (py - v1x) * (v3y - v1y)
        rowml = (iyf >= xminf) & (iyf < xmaxf)
        zrl = jnp.where(rowml, bz * py + az, _NEG_INF)
        rstack = jnp.concatenate([rxA, rxB, rxC, zrl], axis=0)
        rT = jnp.transpose(rstack, (1, 0))

        for c in range(NCH):
            sl = slice(c * CHR, (c + 1) * CHR)
            txA = rT[sl, 0:1]
            txB = rT[sl, 1:2]
            txC = rT[sl, 2:3]
            zr = rT[sl, 3:4]

            pAB = txA - tyA
            pCB = txB - tyB
            pCA = txC - tyC
            emin = jnp.minimum(jnp.minimum(pAB, pCB), pCA)
            z = zr + zc
            zcand = jnp.where(emin > 0.0, z, _NEG_INF)
            zb_ref[sl, :] = jnp.maximum(zb_ref[sl, :], zcand)
        return carry

    jax.lax.fori_loop(0, NT, body, 0)

    zb = zb_ref[...]
    px = linr_ref[...]
    hit = zb > zmin
    ox_ref[...] = jnp.where(hit, px, 0.0)
    oy_ref[...] = jnp.where(hit, py, 0.0)
    oz_ref[...] = jnp.where(hit, zb, 0.0)
    oa_ref[...] = jnp.where(hit, 1.0, 0.0)


@functools.partial(jax.jit)
def kernel(tris):
    tris = tris.astype(jnp.float32)
    zmin = tris.reshape(-1, 3).min(axis=0)[-1]
    lin = jnp.linspace(-1.0, 1.0, SZ, dtype=jnp.float32)

    v1 = tris[:, 0, :]
    v2 = tris[:, 1, :]
    v3 = tris[:, 2, :]
    w = (v2[:, 0] - v1[:, 0]) * (v3[:, 1] - v1[:, 1]) - \
        (v2[:, 1] - v1[:, 1]) * (v3[:, 0] - v1[:, 0])
    valid = jnp.logical_not(w < 1e-9)
    invw = 1.0 / jnp.where(valid, w, 1.0)

    d1 = v1[:, 2] - v3[:, 2]
    d2 = v2[:, 2] - v3[:, 2]
    bCB = v2[:, 1] - v3[:, 1]
    eCB = v2[:, 0] - v3[:, 0]
    aCB = -v3[:, 0] * bCB + v3[:, 1] * eCB
    bCA = v3[:, 1] - v1[:, 1]
    eCA = v3[:, 0] - v1[:, 0]
    aCA = -v1[:, 0] * bCA + v1[:, 1] * eCA
    az = v3[:, 2] + (aCB * d1 + aCA * d2) * invw
    bz = (bCB * d1 + bCA * d2) * invw
    cz = (-eCB * d1 - eCA * d2) * invw

    tri2d = tris[:, :, :2]
    aabb_min = tri2d.min(axis=1)
    aabb_max = tri2d.max(axis=1)

    def a2i(v):
        return jnp.trunc((jnp.clip(v, -1.0, 1.0) + 1.0) / 2.0 * SZ)

    xminf = a2i(aabb_min[:, 0])
    yminf = a2i(aabb_min[:, 1])
    xmaxf = a2i(aabb_max[:, 0])
    ymaxf = a2i(aabb_max[:, 1])
    xminf = jnp.where(valid, xminf, 0.0)
    xmaxf = jnp.where(valid, xmaxf, 0.0)

    zmin_row = jnp.full((NT,), zmin, dtype=jnp.float32)
    td = jnp.stack([
        v1[:, 0], v1[:, 1], v2[:, 0], v2[:, 1], v3[:, 0], v3[:, 1],
        az, bz, cz,
        xminf, xmaxf, yminf, ymaxf,
        zmin_row,
    ], axis=0)

    linr = lin[:, None]
    linc = lin[None, :]

    out_sds = jax.ShapeDtypeStruct((SZ, SZ), jnp.float32)
    ox, oy, oz, oa = pl.pallas_call(
        _raster_kernel,
        in_specs=[
            pl.BlockSpec(memory_space=pltpu.SMEM),
            pl.BlockSpec(memory_space=pltpu.VMEM),
            pl.BlockSpec(memory_space=pltpu.VMEM),
        ],
        out_specs=[
            pl.BlockSpec(memory_space=pltpu.VMEM),
            pl.BlockSpec(memory_space=pltpu.VMEM),
            pl.BlockSpec(memory_space=pltpu.VMEM),
            pl.BlockSpec(memory_space=pltpu.VMEM),
        ],
        out_shape=[out_sds, out_sds, out_sds, out_sds],
        scratch_shapes=[pltpu.VMEM((SZ, SZ), jnp.float32)],
    )(td, linr, linc)

    return jnp.stack([ox, oy, oz, oa], axis=-1)


# speedup vs baseline: 1.3487x; 1.3487x over previous
import functools

import jax
import jax.numpy as jnp
from jax.experimental import pallas as pl
from jax.experimental.pallas import tpu as pltpu

SZ = 512
NT = 256
CHR = 16
NCH = SZ // CHR

_NEG_INF = float("-inf")


def _raster_kernel(td_ref, linr_ref, linc_ref,
                   ox_ref, oy_ref, oz_ref, oa_ref, zb_ref):
    py = linc_ref[...]
    iyf = jax.lax.broadcasted_iota(jnp.int32, (1, SZ), 1).astype(jnp.float32)

    zmin = td_ref[13, 0]
    zb_ref[...] = jnp.full((SZ, SZ), zmin, dtype=jnp.float32)

    def body(t, carry):
        v1x = td_ref[0, t]
        v1y = td_ref[1, t]
        v2x = td_ref[2, t]
        v2y = td_ref[3, t]
        v3x = td_ref[4, t]
        v3y = td_ref[5, t]
        az = td_ref[6, t]
        bz = td_ref[7, t]
        cz = td_ref[8, t]
        xminf = td_ref[9, t]
        xmaxf = td_ref[10, t]
        yminf = td_ref[11, t]
        ymaxf = td_ref[12, t]

        tyA = (py - v2y) * (v1x - v2x)
        tyB = (py - v3y) * (v2x - v3x)
        tyC = (py - v1y) * (v3x - v1x)
        colm = (iyf >= yminf) & (iyf < ymaxf)
        zc = jnp.where(colm, cz * py, _NEG_INF)

        for c in range(NCH):
            sl = slice(c * CHR, (c + 1) * CHR)
            pxc = linr_ref[sl, :]
            ixf = (c * CHR
                   + jax.lax.broadcasted_iota(jnp.int32, (CHR, 1), 0)
                   ).astype(jnp.float32)
            txA = (pxc - v2x) * (v1y - v2y)
            txB = (pxc - v3x) * (v2y - v3y)
            txC = (pxc - v1x) * (v3y - v1y)
            rowm = (ixf >= xminf) & (ixf < xmaxf)
            zr = jnp.where(rowm, bz * pxc + az, _NEG_INF)

            pAB = txA - tyA
            pCB = txB - tyB
            pCA = txC - tyC
            emin = jnp.minimum(jnp.minimum(pAB, pCB), pCA)
            z = zr + zc
            zcand = jnp.where(emin > 0.0, z, _NEG_INF)
            zb_ref[sl, :] = jnp.maximum(zb_ref[sl, :], zcand)
        return carry

    jax.lax.fori_loop(0, NT, body, 0)

    zb = zb_ref[...]
    px = linr_ref[...]
    hit = zb > zmin
    ox_ref[...] = jnp.where(hit, px, 0.0)
    oy_ref[...] = jnp.where(hit, py, 0.0)
    oz_ref[...] = jnp.where(hit, zb, 0.0)
    oa_ref[...] = jnp.where(hit, 1.0, 0.0)


@functools.partial(jax.jit)
def kernel(tris):
    tris = tris.astype(jnp.float32)
    zmin = tris.reshape(-1, 3).min(axis=0)[-1]
    lin = jnp.linspace(-1.0, 1.0, SZ, dtype=jnp.float32)

    v1 = tris[:, 0, :]
    v2 = tris[:, 1, :]
    v3 = tris[:, 2, :]
    w = (v2[:, 0] - v1[:, 0]) * (v3[:, 1] - v1[:, 1]) - \
        (v2[:, 1] - v1[:, 1]) * (v3[:, 0] - v1[:, 0])
    valid = jnp.logical_not(w < 1e-9)
    invw = 1.0 / jnp.where(valid, w, 1.0)

    d1 = v1[:, 2] - v3[:, 2]
    d2 = v2[:, 2] - v3[:, 2]
    bCB = v2[:, 1] - v3[:, 1]
    eCB = v2[:, 0] - v3[:, 0]
    aCB = -v3[:, 0] * bCB + v3[:, 1] * eCB
    bCA = v3[:, 1] - v1[:, 1]
    eCA = v3[:, 0] - v1[:, 0]
    aCA = -v1[:, 0] * bCA + v1[:, 1] * eCA
    az = v3[:, 2] + (aCB * d1 + aCA * d2) * invw
    bz = (bCB * d1 + bCA * d2) * invw
    cz = (-eCB * d1 - eCA * d2) * invw

    tri2d = tris[:, :, :2]
    aabb_min = tri2d.min(axis=1)
    aabb_max = tri2d.max(axis=1)

    def a2i(v):
        return jnp.trunc((jnp.clip(v, -1.0, 1.0) + 1.0) / 2.0 * SZ)

    xminf = a2i(aabb_min[:, 0])
    yminf = a2i(aabb_min[:, 1])
    xmaxf = a2i(aabb_max[:, 0])
    ymaxf = a2i(aabb_max[:, 1])
    xminf = jnp.where(valid, xminf, 0.0)
    xmaxf = jnp.where(valid, xmaxf, 0.0)

    zmin_row = jnp.full((NT,), zmin, dtype=jnp.float32)
    td = jnp.stack([
        v1[:, 0], v1[:, 1], v2[:, 0], v2[:, 1], v3[:, 0], v3[:, 1],
        az, bz, cz,
        xminf, xmaxf, yminf, ymaxf,
        zmin_row,
    ], axis=0)

    linr = lin[:, None]
    linc = lin[None, :]

    out_sds = jax.ShapeDtypeStruct((SZ, SZ), jnp.float32)
    ox, oy, oz, oa = pl.pallas_call(
        _raster_kernel,
        in_specs=[
            pl.BlockSpec(memory_space=pltpu.SMEM),
            pl.BlockSpec(memory_space=pltpu.VMEM),
            pl.BlockSpec(memory_space=pltpu.VMEM),
        ],
        out_specs=[
            pl.BlockSpec(memory_space=pltpu.VMEM),
            pl.BlockSpec(memory_space=pltpu.VMEM),
            pl.BlockSpec(memory_space=pltpu.VMEM),
            pl.BlockSpec(memory_space=pltpu.VMEM),
        ],
        out_shape=[out_sds, out_sds, out_sds, out_sds],
        scratch_shapes=[pltpu.VMEM((SZ, SZ), jnp.float32)],
    )(td, linr, linc)

    return jnp.stack([ox, oy, oz, oa], axis=-1)
